# baseline (speedup 1.0000x reference)
import jax
import jax.numpy as jnp
from jax import lax
from jax.experimental import pallas as pl
from jax.experimental.pallas import tpu as pltpu
from jax.experimental.pallas import tpu_sc as plsc

N_NODES = 10000
D_FEAT = 128
N_EDGES = 320000
ALPHA = 0.1
NUM_P = 4

NC = 2
NS = 16
L = 16

F = D_FEAT // NC
B = 128
GE = 16
NB = 160
NG = NB // GE
ET = NB * B
E_PAD = ET * NS
N_PAD = 10240
NR = N_PAD // NS
CH = 5
RC = NR // CH

_W_SCALE = 1.0 - ALPHA
_INV = 1.0 / _W_SCALE
_A_SCALE = ALPHA / _W_SCALE
_OUT_SCALE = _W_SCALE / NUM_P


def _bcast(x):
    return jnp.full((L,), x, jnp.int32)


def _sc_body(s0_h, src_h, dst_h, w_h, out_h,
             p0, p1, srcs, dsts, wbuf, ra, rb,
             sg1, sg2, sg3, ss1, ss2, qa, qb, pa, pb):
    c_id = lax.axis_index("c")
    s_id = lax.axis_index("s")
    row0 = s_id * NR

    for ch in range(CH):
        r0 = row0 + ch * RC
        pltpu.sync_copy(s0_h.at[c_id, pl.ds(r0, RC), :], ra)

        def _scale_t0(r, carry):
            for k in range(F // L):
                ra[r, pl.ds(k * L, L)] = ra[r, pl.ds(k * L, L)] * _INV
            return carry
        lax.fori_loop(0, RC, _scale_t0, 0)
        pltpu.sync_copy(ra, p0.at[pl.ds(r0, RC)])

        def _scale_a(r, carry):
            for k in range(F // L):
                ra[r, pl.ds(k * L, L)] = ra[r, pl.ds(k * L, L)] * ALPHA
            return carry
        lax.fori_loop(0, RC, _scale_a, 0)
        pltpu.sync_copy(ra, p1.at[pl.ds(r0, RC)])
    plsc.subcore_barrier()

    bufs = (p0, p1)
    for it in range(NUM_P):
        p_prev = bufs[it % 2]
        p_cur = bufs[(it + 1) % 2]

        def _group(gi, carry):
            e0 = pl.multiple_of(s_id * NB + gi * GE, 8)
            c1 = pltpu.async_copy(src_h.at[pl.ds(e0, GE)], srcs, sg1)
            c2 = pltpu.async_copy(dst_h.at[pl.ds(e0, GE)], dsts, sg2)
            c3 = pltpu.async_copy(w_h.at[pl.ds(e0, GE)], wbuf, sg3)
            c1.wait()
            c2.wait()
            c3.wait()

            def _mul(b, rbuf):
                def _wgrp(g, gcarry):
                    wv16 = wbuf[b, pl.ds(g * L, L)] * _W_SCALE
                    e0 = g * L
                    for i in range(L):
                        e = e0 + i
                        wv = wv16.at[_bcast(i)].get(
                            mode='promise_in_bounds')
                        for k in range(F // L):
                            rbuf[e, pl.ds(k * L, L)] = (
                                rbuf[e, pl.ds(k * L, L)] * wv)
                    return gcarry
                lax.fori_loop(0, B // L, _wgrp, 0)

            def _pairblk(q, bcarry):
                b0 = q * 2
                b1 = b0 + 1
                g0 = pltpu.async_copy(p_prev.at[srcs.at[b0]], ra, qa)
                g1 = pltpu.async_copy(p_prev.at[srcs.at[b1]], rb, qb)
                g0.wait()
                _mul(b0, ra)
                s0 = pltpu.async_copy(ra, p_cur.at[dsts.at[b0]], pa,
                                      add=True)
                g1.wait()
                _mul(b1, rb)
                s1 = pltpu.async_copy(rb, p_cur.at[dsts.at[b1]], pb,
                                      add=True)
                s0.wait()
                s1.wait()
                return bcarry
            lax.fori_loop(0, GE // 2, _pairblk, 0)
            return carry
        lax.fori_loop(0, NG, _group, 0)
        plsc.subcore_barrier()

        pend_out = None
        pend_pp = None
        for ch in range(CH):
            r0 = row0 + ch * RC
            if pend_out is not None:
                pend_out.wait()
            c1 = pltpu.async_copy(p_cur.at[pl.ds(r0, RC)], ra, sg1)
            c2 = None
            if it > 0:
                if pend_pp is not None:
                    pend_pp.wait()
                    pend_pp = None
                c2 = pltpu.async_copy(
                    out_h.at[c_id, pl.ds(r0, RC), :], rb, sg2)
            c1.wait()
            if c2 is not None:
                c2.wait()

            def _acc(r, carry):
                for k in range(F // L):
                    v = ra[r, pl.ds(k * L, L)]
                    if it > 0:
                        v = v + rb[r, pl.ds(k * L, L)]
                    if it == NUM_P - 1:
                        v = v * _OUT_SCALE
                    ra[r, pl.ds(k * L, L)] = v
                return carry
            lax.fori_loop(0, RC, _acc, 0)
            pend_out = pltpu.async_copy(
                ra, out_h.at[c_id, pl.ds(r0, RC), :], ss1)

            if it < NUM_P - 1:
                if pend_pp is not None:
                    pend_pp.wait()
                    pend_pp = None
                c3 = pltpu.async_copy(
                    s0_h.at[c_id, pl.ds(r0, RC), :], rb, sg3)
                c3.wait()

                def _reinit(r, carry):
                    for k in range(F // L):
                        rb[r, pl.ds(k * L, L)] = (
                            rb[r, pl.ds(k * L, L)] * _A_SCALE)
                    return carry
                lax.fori_loop(0, RC, _reinit, 0)
                pend_pp = pltpu.async_copy(
                    rb, p_prev.at[pl.ds(r0, RC)], ss2)
        if pend_out is not None:
            pend_out.wait()
        if pend_pp is not None:
            pend_pp.wait()
        if it < NUM_P - 1:
            plsc.subcore_barrier()


def kernel(s0, edge_index, edge_weight):
    src = edge_index[0].astype(jnp.int32)
    dst = edge_index[1].astype(jnp.int32)
    w = edge_weight.astype(jnp.float32)
    pad = E_PAD - N_EDGES
    src = jnp.pad(src, (0, pad)).reshape(NS * NB, B)
    dst = jnp.pad(dst, (0, pad)).reshape(NS * NB, B)
    w = jnp.pad(w, (0, pad)).reshape(NS * NB, B)
    s0p = jnp.pad(s0, ((0, N_PAD - N_NODES), (0, 0)))
    s0s = jnp.stack([s0p[:, :F], s0p[:, F:]], axis=0)

    mesh = plsc.VectorSubcoreMesh(core_axis_name="c", subcore_axis_name="s")
    run = pl.kernel(
        _sc_body,
        out_type=jax.ShapeDtypeStruct((NC, N_PAD, F), jnp.float32),
        mesh=mesh,
        scratch_types=[
            pltpu.VMEM_SHARED((N_PAD, F), jnp.float32),
            pltpu.VMEM_SHARED((N_PAD, F), jnp.float32),
            pltpu.VMEM((GE, B), jnp.int32),
            pltpu.VMEM((GE, B), jnp.int32),
            pltpu.VMEM((GE, B), jnp.float32),
            pltpu.VMEM((B, F), jnp.float32),
            pltpu.VMEM((B, F), jnp.float32),
            pltpu.SemaphoreType.DMA,
            pltpu.SemaphoreType.DMA,
            pltpu.SemaphoreType.DMA,
            pltpu.SemaphoreType.DMA,
            pltpu.SemaphoreType.DMA,
            pltpu.SemaphoreType.DMA,
            pltpu.SemaphoreType.DMA,
            pltpu.SemaphoreType.DMA,
            pltpu.SemaphoreType.DMA,
        ],
    )
    o = run(s0s, src, dst, w)
    return jnp.concatenate([o[0, :N_NODES], o[1, :N_NODES]], axis=1)


# speedup vs baseline: 1.0814x; 1.0814x over previous
import jax
import jax.numpy as jnp
from jax import lax
from jax.experimental import pallas as pl
from jax.experimental.pallas import tpu as pltpu
from jax.experimental.pallas import tpu_sc as plsc

N_NODES = 10000
D_FEAT = 128
N_EDGES = 320000
ALPHA = 0.1
NUM_P = 4

NC = 2
NS = 16
L = 16

F = D_FEAT // NC
B = 128
GE = 32
NB = 160
NG = NB // GE
ET = NB * B
E_PAD = ET * NS
N_PAD = 10240
NR = N_PAD // NS
CH = 5
RC = NR // CH

_W_SCALE = 1.0 - ALPHA
_INV = 1.0 / _W_SCALE
_A_SCALE = ALPHA / _W_SCALE
_OUT_SCALE = _W_SCALE / NUM_P


def _bcast(x):
    return jnp.full((L,), x, jnp.int32)


def _sc_body(s0_h, src_h, dst_h, w_h, out_h,
             p0, p1, srcs, dsts, wbuf, rows, tmp,
             sg1, sg2, sg3, ss1, ss2):
    c_id = lax.axis_index("c")
    s_id = lax.axis_index("s")
    row0 = s_id * NR

    for ch in range(CH):
        r0 = row0 + ch * RC
        pltpu.sync_copy(s0_h.at[c_id, pl.ds(r0, RC), :], tmp)

        def _scale_t0(r, carry):
            for k in range(F // L):
                tmp[r, pl.ds(k * L, L)] = tmp[r, pl.ds(k * L, L)] * _INV
            return carry
        lax.fori_loop(0, RC, _scale_t0, 0)
        pltpu.sync_copy(tmp, p0.at[pl.ds(r0, RC)])

        def _scale_a(r, carry):
            for k in range(F // L):
                tmp[r, pl.ds(k * L, L)] = tmp[r, pl.ds(k * L, L)] * ALPHA
            return carry
        lax.fori_loop(0, RC, _scale_a, 0)
        pltpu.sync_copy(tmp, p1.at[pl.ds(r0, RC)])
    plsc.subcore_barrier()

    bufs = (p0, p1)
    for it in range(NUM_P):
        p_prev = bufs[it % 2]
        p_cur = bufs[(it + 1) % 2]

        def _group(gi, carry):
            e0 = pl.multiple_of(s_id * NB + gi * GE, 8)
            c1 = pltpu.async_copy(src_h.at[pl.ds(e0, GE)], srcs, sg1)
            c2 = pltpu.async_copy(dst_h.at[pl.ds(e0, GE)], dsts, sg2)
            c3 = pltpu.async_copy(w_h.at[pl.ds(e0, GE)], wbuf, sg3)
            c1.wait()
            c2.wait()
            c3.wait()

            def _block(b, bcarry):
                pltpu.sync_copy(p_prev.at[srcs.at[b]], rows)

                def _wgrp(g, gcarry):
                    wv16 = wbuf[b, pl.ds(g * L, L)] * _W_SCALE
                    e0 = g * L
                    for i in range(L):
                        e = e0 + i
                        wv = wv16.at[_bcast(i)].get(
                            mode='promise_in_bounds')
                        for k in range(F // L):
                            rows[e, pl.ds(k * L, L)] = (
                                rows[e, pl.ds(k * L, L)] * wv)
                    return gcarry
                lax.fori_loop(0, B // L, _wgrp, 0)
                pltpu.sync_copy(rows, p_cur.at[dsts.at[b]], add=True)
                return bcarry
            lax.fori_loop(0, GE, _block, 0)
            return carry
        lax.fori_loop(0, NG, _group, 0)
        plsc.subcore_barrier()

        pend_out = None
        pend_pp = None
        for ch in range(CH):
            r0 = row0 + ch * RC
            if pend_out is not None:
                pend_out.wait()
            c1 = pltpu.async_copy(p_cur.at[pl.ds(r0, RC)], tmp, sg1)
            c2 = None
            if it > 0:
                if pend_pp is not None:
                    pend_pp.wait()
                    pend_pp = None
                c2 = pltpu.async_copy(
                    out_h.at[c_id, pl.ds(r0, RC), :], rows, sg2)
            c1.wait()
            if c2 is not None:
                c2.wait()

            def _acc(r, carry):
                for k in range(F // L):
                    v = tmp[r, pl.ds(k * L, L)]
                    if it > 0:
                        v = v + rows[r, pl.ds(k * L, L)]
                    if it == NUM_P - 1:
                        v = v * _OUT_SCALE
                    tmp[r, pl.ds(k * L, L)] = v
                return carry
            lax.fori_loop(0, RC, _acc, 0)
            pend_out = pltpu.async_copy(
                tmp, out_h.at[c_id, pl.ds(r0, RC), :], ss1)

            if it < NUM_P - 1:
                if pend_pp is not None:
                    pend_pp.wait()
                    pend_pp = None
                c3 = pltpu.async_copy(
                    s0_h.at[c_id, pl.ds(r0, RC), :], rows, sg3)
                c3.wait()

                def _reinit(r, carry):
                    for k in range(F // L):
                        rows[r, pl.ds(k * L, L)] = (
                            rows[r, pl.ds(k * L, L)] * _A_SCALE)
                    return carry
                lax.fori_loop(0, RC, _reinit, 0)
                pend_pp = pltpu.async_copy(
                    rows, p_prev.at[pl.ds(r0, RC)], ss2)
        if pend_out is not None:
            pend_out.wait()
        if pend_pp is not None:
            pend_pp.wait()
        if it < NUM_P - 1:
            plsc.subcore_barrier()


def kernel(s0, edge_index, edge_weight):
    src = edge_index[0].astype(jnp.int32)
    dst = edge_index[1].astype(jnp.int32)
    w = edge_weight.astype(jnp.float32)
    pad = E_PAD - N_EDGES
    src = jnp.pad(src, (0, pad)).reshape(NS * NB, B)
    dst = jnp.pad(dst, (0, pad)).reshape(NS * NB, B)
    w = jnp.pad(w, (0, pad)).reshape(NS * NB, B)
    s0p = jnp.pad(s0, ((0, N_PAD - N_NODES), (0, 0)))
    s0s = jnp.stack([s0p[:, :F], s0p[:, F:]], axis=0)

    mesh = plsc.VectorSubcoreMesh(core_axis_name="c", subcore_axis_name="s")
    run = pl.kernel(
        _sc_body,
        out_type=jax.ShapeDtypeStruct((NC, N_PAD, F), jnp.float32),
        mesh=mesh,
        scratch_types=[
            pltpu.VMEM_SHARED((N_PAD, F), jnp.float32),
            pltpu.VMEM_SHARED((N_PAD, F), jnp.float32),
            pltpu.VMEM((GE, B), jnp.int32),
            pltpu.VMEM((GE, B), jnp.int32),
            pltpu.VMEM((GE, B), jnp.float32),
            pltpu.VMEM((B, F), jnp.float32),
            pltpu.VMEM((RC, F), jnp.float32),
            pltpu.SemaphoreType.DMA,
            pltpu.SemaphoreType.DMA,
            pltpu.SemaphoreType.DMA,
            pltpu.SemaphoreType.DMA,
            pltpu.SemaphoreType.DMA,
        ],
    )
    o = run(s0s, src, dst, w)
    return jnp.concatenate([o[0, :N_NODES], o[1, :N_NODES]], axis=1)


# speedup vs baseline: 1.0875x; 1.0056x over previous
import jax
import jax.numpy as jnp
from jax import lax
from jax.experimental import pallas as pl
from jax.experimental.pallas import tpu as pltpu
from jax.experimental.pallas import tpu_sc as plsc

N_NODES = 10000
D_FEAT = 128
N_EDGES = 320000
ALPHA = 0.1
NUM_P = 4

NC = 2
NS = 16
L = 16

F = D_FEAT // NC
B = 128
GE = 40
NB = 160
NG = NB // GE
ET = NB * B
E_PAD = ET * NS
N_PAD = 10240
NR = N_PAD // NS
CH = 5
RC = NR // CH

_W_SCALE = 1.0 - ALPHA
_INV = 1.0 / _W_SCALE
_A_SCALE = ALPHA / _W_SCALE
_OUT_SCALE = _W_SCALE / NUM_P


def _bcast(x):
    return jnp.full((L,), x, jnp.int32)


def _sc_body(s0_h, src_h, dst_h, w_h, out_h,
             p0, p1, srcs, dsts, wbuf, rows, tmp,
             sg1, sg2, sg3, ss1, ss2):
    c_id = lax.axis_index("c")
    s_id = lax.axis_index("s")
    row0 = s_id * NR

    for ch in range(CH):
        r0 = row0 + ch * RC
        pltpu.sync_copy(s0_h.at[c_id, pl.ds(r0, RC), :], tmp)

        def _scale_t0(r, carry):
            for k in range(F // L):
                tmp[r, pl.ds(k * L, L)] = tmp[r, pl.ds(k * L, L)] * _INV
            return carry
        lax.fori_loop(0, RC, _scale_t0, 0)
        pltpu.sync_copy(tmp, p0.at[pl.ds(r0, RC)])

        def _scale_a(r, carry):
            for k in range(F // L):
                tmp[r, pl.ds(k * L, L)] = tmp[r, pl.ds(k * L, L)] * ALPHA
            return carry
        lax.fori_loop(0, RC, _scale_a, 0)
        pltpu.sync_copy(tmp, p1.at[pl.ds(r0, RC)])
    plsc.subcore_barrier()

    bufs = (p0, p1)
    for it in range(NUM_P):
        p_prev = bufs[it % 2]
        p_cur = bufs[(it + 1) % 2]

        def _group(gi, carry):
            e0 = pl.multiple_of(s_id * NB + gi * GE, 8)
            c1 = pltpu.async_copy(src_h.at[pl.ds(e0, GE)], srcs, sg1)
            c2 = pltpu.async_copy(dst_h.at[pl.ds(e0, GE)], dsts, sg2)
            c3 = pltpu.async_copy(w_h.at[pl.ds(e0, GE)], wbuf, sg3)
            c1.wait()
            c2.wait()
            c3.wait()

            def _block(b, bcarry):
                pltpu.sync_copy(p_prev.at[srcs.at[b]], rows)

                def _wgrp(g, gcarry):
                    wv16 = wbuf[b, pl.ds(g * L, L)] * _W_SCALE
                    e0 = g * L
                    for i in range(L):
                        e = e0 + i
                        wv = wv16.at[_bcast(i)].get(
                            mode='promise_in_bounds')
                        for k in range(F // L):
                            rows[e, pl.ds(k * L, L)] = (
                                rows[e, pl.ds(k * L, L)] * wv)
                    return gcarry
                lax.fori_loop(0, B // L, _wgrp, 0)
                pltpu.sync_copy(rows, p_cur.at[dsts.at[b]], add=True)
                return bcarry
            lax.fori_loop(0, GE, _block, 0)
            return carry
        lax.fori_loop(0, NG, _group, 0)
        plsc.subcore_barrier()

        pend_out = None
        pend_pp = None
        for ch in range(CH):
            r0 = row0 + ch * RC
            if pend_out is not None:
                pend_out.wait()
            c1 = pltpu.async_copy(p_cur.at[pl.ds(r0, RC)], tmp, sg1)
            c2 = None
            if it > 0:
                if pend_pp is not None:
                    pend_pp.wait()
                    pend_pp = None
                c2 = pltpu.async_copy(
                    out_h.at[c_id, pl.ds(r0, RC), :], rows, sg2)
            c1.wait()
            if c2 is not None:
                c2.wait()

            def _acc(r, carry):
                for k in range(F // L):
                    v = tmp[r, pl.ds(k * L, L)]
                    if it > 0:
                        v = v + rows[r, pl.ds(k * L, L)]
                    if it == NUM_P - 1:
                        v = v * _OUT_SCALE
                    tmp[r, pl.ds(k * L, L)] = v
                return carry
            lax.fori_loop(0, RC, _acc, 0)
            pend_out = pltpu.async_copy(
                tmp, out_h.at[c_id, pl.ds(r0, RC), :], ss1)

            if it < NUM_P - 1:
                if pend_pp is not None:
                    pend_pp.wait()
                    pend_pp = None
                c3 = pltpu.async_copy(
                    s0_h.at[c_id, pl.ds(r0, RC), :], rows, sg3)
                c3.wait()

                def _reinit(r, carry):
                    for k in range(F // L):
                        rows[r, pl.ds(k * L, L)] = (
                            rows[r, pl.ds(k * L, L)] * _A_SCALE)
                    return carry
                lax.fori_loop(0, RC, _reinit, 0)
                pend_pp = pltpu.async_copy(
                    rows, p_prev.at[pl.ds(r0, RC)], ss2)
        if pend_out is not None:
            pend_out.wait()
        if pend_pp is not None:
            pend_pp.wait()
        if it < NUM_P - 1:
            plsc.subcore_barrier()


def kernel(s0, edge_index, edge_weight):
    src = edge_index[0].astype(jnp.int32)
    dst = edge_index[1].astype(jnp.int32)
    w = edge_weight.astype(jnp.float32)
    pad = E_PAD - N_EDGES
    src = jnp.pad(src, (0, pad)).reshape(NS * NB, B)
    dst = jnp.pad(dst, (0, pad)).reshape(NS * NB, B)
    w = jnp.pad(w, (0, pad)).reshape(NS * NB, B)
    s0p = jnp.pad(s0, ((0, N_PAD - N_NODES), (0, 0)))
    s0s = jnp.stack([s0p[:, :F], s0p[:, F:]], axis=0)

    mesh = plsc.VectorSubcoreMesh(core_axis_name="c", subcore_axis_name="s")
    run = pl.kernel(
        _sc_body,
        out_type=jax.ShapeDtypeStruct((NC, N_PAD, F), jnp.float32),
        mesh=mesh,
        scratch_types=[
            pltpu.VMEM_SHARED((N_PAD, F), jnp.float32),
            pltpu.VMEM_SHARED((N_PAD, F), jnp.float32),
            pltpu.VMEM((GE, B), jnp.int32),
            pltpu.VMEM((GE, B), jnp.int32),
            pltpu.VMEM((GE, B), jnp.float32),
            pltpu.VMEM((B, F), jnp.float32),
            pltpu.VMEM((RC, F), jnp.float32),
            pltpu.SemaphoreType.DMA,
            pltpu.SemaphoreType.DMA,
            pltpu.SemaphoreType.DMA,
            pltpu.SemaphoreType.DMA,
            pltpu.SemaphoreType.DMA,
        ],
    )
    o = run(s0s, src, dst, w)
    return jnp.concatenate([o[0, :N_NODES], o[1, :N_NODES]], axis=1)
